# baseline (speedup 1.0000x reference)
import functools

import jax
import jax.numpy as jnp
from jax import lax
from jax.experimental import pallas as pl
from jax.experimental.pallas import tpu as pltpu
from jax.experimental.pallas import tpu_sc as plsc

N = 10000
E = 320000
D = 128
NRA = 10240

NC = 2
NS = 16
NW = NC * NS
B = 125
KB = 8
CPT = 80
NG = CPT // KB
NG2 = NG // 2
NCH = NW * CPT
RPT = NRA // NS

_MESH = plsc.VectorSubcoreMesh(core_axis_name="c", subcore_axis_name="s")


def _worker_id():
    return lax.axis_index("s") * NC + lax.axis_index("c")


def _gather_scatter_group(table, sb, db, acc_sh, vals, gsem, ssem):
    nbuf = len(vals)
    h_g = [None] * nbuf
    h_s = [None] * nbuf
    state = {"started": 0}

    def start_one():
        j = state["started"]
        p = j % nbuf
        if h_s[p] is not None:
            h_s[p].wait()
        h_g[p] = pltpu.async_copy(table.at[sb.at[j]], vals[p], gsem[p])
        state["started"] = j + 1

    for _ in range(min(nbuf, KB)):
        start_one()
    for k in range(KB):
        p = k % nbuf
        h_g[p].wait()
        h_s[p] = pltpu.async_copy(vals[p], acc_sh.at[db.at[k]], ssem[p],
                                  add=True)
        if state["started"] < KB:
            start_one()
    for p in range(nbuf):
        if h_s[p] is not None:
            h_s[p].wait()


def _make_edge_pass(d_feat, stage_table, nbuf):
    if d_feat == 1:
        tshape = (NRA,)
        ashape = (NRA,)
        vshape = (B,)
        zshape = (RPT,)
    else:
        tshape = (N, d_feat)
        ashape = (NRA, d_feat)
        vshape = (B, d_feat)
        zshape = (RPT, d_feat)

    scratch = (
        [pltpu.VMEM((KB, B), jnp.int32)] * 4
        + [pltpu.VMEM(vshape, jnp.float32)] * nbuf
        + [pltpu.VMEM_SHARED(ashape, jnp.float32)]
        + ([pltpu.VMEM_SHARED(tshape, jnp.float32)] if stage_table else [])
        + [pltpu.SemaphoreType.DMA] * (2 * nbuf + 2)
    )

    @functools.partial(
        pl.kernel,
        mesh=_MESH,
        out_type=jax.ShapeDtypeStruct((NC,) + ashape, jnp.float32),
        scratch_types=scratch,
    )
    def edge_pass(table_hbm, srcm_hbm, dstm_hbm, zeros_hbm, out_hbm,
                  sb0, sb1, db0, db1, *rest):
        vals = tuple(rest[:nbuf])
        acc_sh = rest[nbuf]
        rest = rest[nbuf + 1:]
        if stage_table:
            tab_sh = rest[0]
            rest = rest[1:]
        else:
            tab_sh = None
        gsem = tuple(rest[:nbuf])
        ssem = tuple(rest[nbuf:2 * nbuf])
        semi0, semi1 = rest[2 * nbuf:]

        c = lax.axis_index("c")
        s = lax.axis_index("s")
        wid = _worker_id()
        r0 = s * RPT
        pltpu.sync_copy(zeros_hbm, acc_sh.at[pl.ds(r0, RPT)])
        if stage_table:
            pltpu.sync_copy(table_hbm.at[pl.ds(r0, RPT)],
                            tab_sh.at[pl.ds(r0, RPT)])
            table = tab_sh
        else:
            table = table_hbm
        plsc.subcore_barrier()

        row0 = wid * CPT

        pltpu.async_copy(srcm_hbm.at[pl.ds(row0, KB)], sb0, semi0)
        pltpu.async_copy(dstm_hbm.at[pl.ds(row0, KB)], db0, semi0)

        def body(t, carry):
            grow1 = row0 + (2 * t + 1) * KB
            hi_s = pltpu.async_copy(srcm_hbm.at[pl.ds(grow1, KB)], sb1, semi1)
            hi_d = pltpu.async_copy(dstm_hbm.at[pl.ds(grow1, KB)], db1, semi1)
            pltpu.make_async_copy(srcm_hbm.at[pl.ds(row0, KB)], sb0, semi0).wait()
            pltpu.make_async_copy(dstm_hbm.at[pl.ds(row0, KB)], db0, semi0).wait()
            _gather_scatter_group(table, sb0, db0, acc_sh, vals, gsem, ssem)

            @pl.when(t < NG2 - 1)
            def _():
                grow2 = row0 + (2 * t + 2) * KB
                pltpu.async_copy(srcm_hbm.at[pl.ds(grow2, KB)], sb0, semi0)
                pltpu.async_copy(dstm_hbm.at[pl.ds(grow2, KB)], db0, semi0)

            hi_s.wait()
            hi_d.wait()
            _gather_scatter_group(table, sb1, db1, acc_sh, vals, gsem, ssem)
            return carry

        lax.fori_loop(0, NG2, body, 0)
        plsc.subcore_barrier()
        pltpu.sync_copy(acc_sh.at[pl.ds(r0, RPT)],
                        out_hbm.at[c].at[pl.ds(r0, RPT)])

    return edge_pass


def _make_deg_pass():
    @functools.partial(
        pl.kernel,
        mesh=_MESH,
        out_type=jax.ShapeDtypeStruct((NC, NRA), jnp.float32),
        scratch_types=[
            pltpu.VMEM((KB, B), jnp.int32),
            pltpu.VMEM((KB, B), jnp.int32),
            pltpu.VMEM((B,), jnp.float32),
            pltpu.VMEM_SHARED((NRA,), jnp.float32),
            pltpu.SemaphoreType.DMA,
            pltpu.SemaphoreType.DMA,
            pltpu.SemaphoreType.DMA,
        ],
    )
    def deg_pass(dstm_hbm, ones_hbm, zeros_hbm, out_hbm, db0, db1, ones_v,
                 acc_sh, ssem, semi0, semi1):
        c = lax.axis_index("c")
        s = lax.axis_index("s")
        wid = _worker_id()
        r0 = s * RPT
        pltpu.sync_copy(zeros_hbm, acc_sh.at[pl.ds(r0, RPT)])
        pltpu.sync_copy(ones_hbm, ones_v)
        plsc.subcore_barrier()

        row0 = wid * CPT

        def scatter_group(db):
            hs = [pltpu.async_copy(ones_v, acc_sh.at[db.at[k]], ssem, add=True)
                  for k in range(KB)]
            for h in hs:
                h.wait()

        pltpu.async_copy(dstm_hbm.at[pl.ds(row0, KB)], db0, semi0)

        def body(t, carry):
            grow1 = row0 + (2 * t + 1) * KB
            hi_d = pltpu.async_copy(dstm_hbm.at[pl.ds(grow1, KB)], db1, semi1)
            pltpu.make_async_copy(dstm_hbm.at[pl.ds(row0, KB)], db0, semi0).wait()
            scatter_group(db0)

            @pl.when(t < NG2 - 1)
            def _():
                grow2 = row0 + (2 * t + 2) * KB
                pltpu.async_copy(dstm_hbm.at[pl.ds(grow2, KB)], db0, semi0)

            hi_d.wait()
            scatter_group(db1)
            return carry

        lax.fori_loop(0, NG2, body, 0)
        plsc.subcore_barrier()
        pltpu.sync_copy(acc_sh.at[pl.ds(r0, RPT)],
                        out_hbm.at[c].at[pl.ds(r0, RPT)])

    return deg_pass


_BR = 1000
_GRID = N // _BR


def _tc_matmul(x, W1):
    def body(x_ref, w_ref, xw_ref):
        xw_ref[...] = jnp.dot(x_ref[...], w_ref[...],
                              preferred_element_type=jnp.float32)

    return pl.pallas_call(
        body,
        grid=(_GRID,),
        in_specs=[
            pl.BlockSpec((_BR, D), lambda i: (i, 0)),
            pl.BlockSpec((D, D), lambda i: (0, 0)),
        ],
        out_specs=pl.BlockSpec((_BR, D), lambda i: (i, 0)),
        out_shape=jax.ShapeDtypeStruct((N, D), jnp.float32),
    )(x, W1)


def _tc_scale(xw, degp):
    def body(xw_ref, degp_ref, xwp_ref, dis_ref):
        deg = degp_ref[0] + degp_ref[1] + 1.0
        dis = lax.rsqrt(deg)
        xwp_ref[...] = dis * xw_ref[...]
        dis_ref[...] = dis

    return pl.pallas_call(
        body,
        grid=(_GRID,),
        in_specs=[
            pl.BlockSpec((_BR, D), lambda i: (i, 0)),
            pl.BlockSpec((NC, _BR, 1), lambda i: (0, i, 0)),
        ],
        out_specs=[
            pl.BlockSpec((_BR, D), lambda i: (i, 0)),
            pl.BlockSpec((_BR, 1), lambda i: (i, 0)),
        ],
        out_shape=[
            jax.ShapeDtypeStruct((N, D), jnp.float32),
            jax.ShapeDtypeStruct((N, 1), jnp.float32),
        ],
    )(xw, degp)


def _tc_layer2(accp, xwp, dis, b1, W2, fc_W):
    def body(accp_ref, xwp_ref, dis_ref, b1_ref, w2_ref, fcw_ref, zp_ref):
        acc = accp_ref[0] + accp_ref[1] + xwp_ref[...]
        h1 = jnp.maximum(dis_ref[...] * acc + b1_ref[...], 0.0)
        w2f = jnp.dot(w2_ref[...], fcw_ref[...], preferred_element_type=jnp.float32)
        z = jnp.dot(h1, w2f, preferred_element_type=jnp.float32)
        zp_ref[...] = dis_ref[...] * z

    return pl.pallas_call(
        body,
        grid=(_GRID,),
        in_specs=[
            pl.BlockSpec((NC, _BR, D), lambda i: (0, i, 0)),
            pl.BlockSpec((_BR, D), lambda i: (i, 0)),
            pl.BlockSpec((_BR, 1), lambda i: (i, 0)),
            pl.BlockSpec((1, D), lambda i: (0, 0)),
            pl.BlockSpec((D, 64), lambda i: (0, 0)),
            pl.BlockSpec((64, 1), lambda i: (0, 0)),
        ],
        out_specs=pl.BlockSpec((_BR, 1), lambda i: (i, 0)),
        out_shape=jax.ShapeDtypeStruct((N, 1), jnp.float32),
    )(accp, xwp, dis, b1, W2, fc_W)


def _tc_final(acc2p, zp, dis, b2, fc_W, fc_b):
    def body(acc2p_ref, zp_ref, dis_ref, b2_ref, fcw_ref, fcb_ref, out_ref):
        t = dis_ref[...] * (acc2p_ref[0] + acc2p_ref[1] + zp_ref[...])
        bias = jnp.dot(b2_ref[...], fcw_ref[...],
                       preferred_element_type=jnp.float32) + fcb_ref[...]
        out_ref[...] = jax.nn.sigmoid(t + bias)

    return pl.pallas_call(
        body,
        grid=(_GRID,),
        in_specs=[
            pl.BlockSpec((NC, _BR, 1), lambda i: (0, i, 0)),
            pl.BlockSpec((_BR, 1), lambda i: (i, 0)),
            pl.BlockSpec((_BR, 1), lambda i: (i, 0)),
            pl.BlockSpec((1, 64), lambda i: (0, 0)),
            pl.BlockSpec((64, 1), lambda i: (0, 0)),
            pl.BlockSpec((1, 1), lambda i: (0, 0)),
        ],
        out_specs=pl.BlockSpec((_BR, 1), lambda i: (i, 0)),
        out_shape=jax.ShapeDtypeStruct((N, 1), jnp.float32),
    )(acc2p, zp, dis, b2, fc_W, fc_b)


def kernel(x, edge_index, W1, b1, W2, b2, fc_W, fc_b):
    srcm = edge_index[0].astype(jnp.int32).reshape(NCH, B)
    dstm = edge_index[1].astype(jnp.int32).reshape(NCH, B)

    zeros2d = jnp.zeros((RPT, D), jnp.float32)
    zeros1d = jnp.zeros((RPT,), jnp.float32)
    ones_b = jnp.ones((B,), jnp.float32)
    x = x.astype(jnp.float32)

    xw = _tc_matmul(x, W1)
    degp = _make_deg_pass()(dstm, ones_b, zeros1d)
    degp = degp.reshape(NC, NRA, 1)

    xwp, dis = _tc_scale(xw, degp)

    accp = _make_edge_pass(D, stage_table=False, nbuf=2)(xwp, srcm, dstm,
                                                         zeros2d)

    zp = _tc_layer2(accp, xwp, dis, b1.reshape(1, D), W2, fc_W)

    zp_pad = jnp.concatenate([zp.reshape(N), jnp.zeros((NRA - N,), jnp.float32)])
    acc2p = _make_edge_pass(1, stage_table=True, nbuf=4)(zp_pad, srcm, dstm,
                                                         zeros1d)
    acc2p = acc2p.reshape(NC, NRA, 1)

    return _tc_final(acc2p, zp, dis, b2.reshape(1, 64), fc_W,
                     fc_b.reshape(1, 1))


# speedup vs baseline: 1.0253x; 1.0253x over previous
import functools

import jax
import jax.numpy as jnp
from jax import lax
from jax.experimental import pallas as pl
from jax.experimental.pallas import tpu as pltpu
from jax.experimental.pallas import tpu_sc as plsc

N = 10000
E = 320000
D = 128
NRA = 10240

NC = 2
NS = 16
NW = NC * NS
B = 125
KB = 8
CPT = 80
NG = CPT // KB
NG2 = NG // 2
NCH = NW * CPT
RPT = NRA // NS

_MESH = plsc.VectorSubcoreMesh(core_axis_name="c", subcore_axis_name="s")


def _worker_id():
    return lax.axis_index("s") * NC + lax.axis_index("c")


def _gather_scatter_2groups(table, sb0, db0, sb1, db1, acc_sh, vals, gsem,
                            ssem, at_idx_wait, at_prefetch):
    nbuf = len(vals)
    n = 2 * KB
    h_g = [None] * nbuf
    h_s = [None] * nbuf
    state = {"started": 0, "waited": False}

    def start_one():
        j = state["started"]
        if j >= KB and not state["waited"]:
            at_idx_wait()
            state["waited"] = True
        sb = sb0 if j < KB else sb1
        p = j % nbuf
        if h_s[p] is not None:
            h_s[p].wait()
        h_g[p] = pltpu.async_copy(table.at[sb.at[j % KB]], vals[p], gsem[p])
        state["started"] = j + 1

    for _ in range(min(nbuf, n)):
        start_one()
    for k in range(n):
        db = db0 if k < KB else db1
        p = k % nbuf
        h_g[p].wait()
        if k == KB - 1:
            at_prefetch()
        h_s[p] = pltpu.async_copy(vals[p], acc_sh.at[db.at[k % KB]], ssem[p],
                                  add=True)
        if state["started"] < n:
            start_one()
    for p in range(nbuf):
        if h_s[p] is not None:
            h_s[p].wait()


def _make_edge_pass(d_feat, stage_table, nbuf):
    if d_feat == 1:
        tshape = (NRA,)
        ashape = (NRA,)
        vshape = (B,)
        zshape = (RPT,)
    else:
        tshape = (N, d_feat)
        ashape = (NRA, d_feat)
        vshape = (B, d_feat)
        zshape = (RPT, d_feat)

    scratch = (
        [pltpu.VMEM((KB, B), jnp.int32)] * 4
        + [pltpu.VMEM(vshape, jnp.float32)] * nbuf
        + [pltpu.VMEM_SHARED(ashape, jnp.float32)]
        + ([pltpu.VMEM_SHARED(tshape, jnp.float32)] if stage_table else [])
        + [pltpu.SemaphoreType.DMA] * (2 * nbuf + 2)
    )

    @functools.partial(
        pl.kernel,
        mesh=_MESH,
        out_type=jax.ShapeDtypeStruct((NC,) + ashape, jnp.float32),
        scratch_types=scratch,
    )
    def edge_pass(table_hbm, srcm_hbm, dstm_hbm, zeros_hbm, out_hbm,
                  sb0, sb1, db0, db1, *rest):
        vals = tuple(rest[:nbuf])
        acc_sh = rest[nbuf]
        rest = rest[nbuf + 1:]
        if stage_table:
            tab_sh = rest[0]
            rest = rest[1:]
        else:
            tab_sh = None
        gsem = tuple(rest[:nbuf])
        ssem = tuple(rest[nbuf:2 * nbuf])
        semi0, semi1 = rest[2 * nbuf:]

        c = lax.axis_index("c")
        s = lax.axis_index("s")
        wid = _worker_id()
        r0 = s * RPT
        pltpu.sync_copy(zeros_hbm, acc_sh.at[pl.ds(r0, RPT)])
        if stage_table:
            pltpu.sync_copy(table_hbm.at[pl.ds(r0, RPT)],
                            tab_sh.at[pl.ds(r0, RPT)])
            table = tab_sh
        else:
            table = table_hbm
        plsc.subcore_barrier()

        row0 = wid * CPT

        pltpu.async_copy(srcm_hbm.at[pl.ds(row0, KB)], sb0, semi0)
        pltpu.async_copy(dstm_hbm.at[pl.ds(row0, KB)], db0, semi0)

        def body(t, carry):
            grow1 = row0 + (2 * t + 1) * KB
            hi_s = pltpu.async_copy(srcm_hbm.at[pl.ds(grow1, KB)], sb1, semi1)
            hi_d = pltpu.async_copy(dstm_hbm.at[pl.ds(grow1, KB)], db1, semi1)
            pltpu.make_async_copy(srcm_hbm.at[pl.ds(row0, KB)], sb0, semi0).wait()
            pltpu.make_async_copy(dstm_hbm.at[pl.ds(row0, KB)], db0, semi0).wait()

            def at_idx_wait():
                hi_s.wait()
                hi_d.wait()

            def at_prefetch():
                @pl.when(t < NG2 - 1)
                def _():
                    grow2 = row0 + (2 * t + 2) * KB
                    pltpu.async_copy(srcm_hbm.at[pl.ds(grow2, KB)], sb0, semi0)
                    pltpu.async_copy(dstm_hbm.at[pl.ds(grow2, KB)], db0, semi0)

            _gather_scatter_2groups(table, sb0, db0, sb1, db1, acc_sh, vals,
                                    gsem, ssem, at_idx_wait, at_prefetch)
            return carry

        lax.fori_loop(0, NG2, body, 0)
        plsc.subcore_barrier()
        pltpu.sync_copy(acc_sh.at[pl.ds(r0, RPT)],
                        out_hbm.at[c].at[pl.ds(r0, RPT)])

    return edge_pass


def _make_deg_pass():
    @functools.partial(
        pl.kernel,
        mesh=_MESH,
        out_type=jax.ShapeDtypeStruct((NC, NRA), jnp.float32),
        scratch_types=[
            pltpu.VMEM((KB, B), jnp.int32),
            pltpu.VMEM((KB, B), jnp.int32),
            pltpu.VMEM((B,), jnp.float32),
            pltpu.VMEM_SHARED((NRA,), jnp.float32),
            pltpu.SemaphoreType.DMA,
            pltpu.SemaphoreType.DMA,
            pltpu.SemaphoreType.DMA,
        ],
    )
    def deg_pass(dstm_hbm, ones_hbm, zeros_hbm, out_hbm, db0, db1, ones_v,
                 acc_sh, ssem, semi0, semi1):
        c = lax.axis_index("c")
        s = lax.axis_index("s")
        wid = _worker_id()
        r0 = s * RPT
        pltpu.sync_copy(zeros_hbm, acc_sh.at[pl.ds(r0, RPT)])
        pltpu.sync_copy(ones_hbm, ones_v)
        plsc.subcore_barrier()

        row0 = wid * CPT

        def scatter_group(db):
            hs = [pltpu.async_copy(ones_v, acc_sh.at[db.at[k]], ssem, add=True)
                  for k in range(KB)]
            for h in hs:
                h.wait()

        pltpu.async_copy(dstm_hbm.at[pl.ds(row0, KB)], db0, semi0)

        def body(t, carry):
            grow1 = row0 + (2 * t + 1) * KB
            hi_d = pltpu.async_copy(dstm_hbm.at[pl.ds(grow1, KB)], db1, semi1)
            pltpu.make_async_copy(dstm_hbm.at[pl.ds(row0, KB)], db0, semi0).wait()
            scatter_group(db0)

            @pl.when(t < NG2 - 1)
            def _():
                grow2 = row0 + (2 * t + 2) * KB
                pltpu.async_copy(dstm_hbm.at[pl.ds(grow2, KB)], db0, semi0)

            hi_d.wait()
            scatter_group(db1)
            return carry

        lax.fori_loop(0, NG2, body, 0)
        plsc.subcore_barrier()
        pltpu.sync_copy(acc_sh.at[pl.ds(r0, RPT)],
                        out_hbm.at[c].at[pl.ds(r0, RPT)])

    return deg_pass


_BR = 1000
_GRID = N // _BR


def _tc_matmul(x, W1):
    def body(x_ref, w_ref, xw_ref):
        xw_ref[...] = jnp.dot(x_ref[...], w_ref[...],
                              preferred_element_type=jnp.float32)

    return pl.pallas_call(
        body,
        grid=(_GRID,),
        in_specs=[
            pl.BlockSpec((_BR, D), lambda i: (i, 0)),
            pl.BlockSpec((D, D), lambda i: (0, 0)),
        ],
        out_specs=pl.BlockSpec((_BR, D), lambda i: (i, 0)),
        out_shape=jax.ShapeDtypeStruct((N, D), jnp.float32),
    )(x, W1)


def _tc_scale(xw, degp):
    def body(xw_ref, degp_ref, xwp_ref, dis_ref):
        deg = degp_ref[0] + degp_ref[1] + 1.0
        dis = lax.rsqrt(deg)
        xwp_ref[...] = dis * xw_ref[...]
        dis_ref[...] = dis

    return pl.pallas_call(
        body,
        grid=(_GRID,),
        in_specs=[
            pl.BlockSpec((_BR, D), lambda i: (i, 0)),
            pl.BlockSpec((NC, _BR, 1), lambda i: (0, i, 0)),
        ],
        out_specs=[
            pl.BlockSpec((_BR, D), lambda i: (i, 0)),
            pl.BlockSpec((_BR, 1), lambda i: (i, 0)),
        ],
        out_shape=[
            jax.ShapeDtypeStruct((N, D), jnp.float32),
            jax.ShapeDtypeStruct((N, 1), jnp.float32),
        ],
    )(xw, degp)


def _tc_layer2(accp, xwp, dis, b1, W2, fc_W):
    def body(accp_ref, xwp_ref, dis_ref, b1_ref, w2_ref, fcw_ref, zp_ref):
        acc = accp_ref[0] + accp_ref[1] + xwp_ref[...]
        h1 = jnp.maximum(dis_ref[...] * acc + b1_ref[...], 0.0)
        w2f = jnp.dot(w2_ref[...], fcw_ref[...], preferred_element_type=jnp.float32)
        z = jnp.dot(h1, w2f, preferred_element_type=jnp.float32)
        zp_ref[...] = dis_ref[...] * z

    return pl.pallas_call(
        body,
        grid=(_GRID,),
        in_specs=[
            pl.BlockSpec((NC, _BR, D), lambda i: (0, i, 0)),
            pl.BlockSpec((_BR, D), lambda i: (i, 0)),
            pl.BlockSpec((_BR, 1), lambda i: (i, 0)),
            pl.BlockSpec((1, D), lambda i: (0, 0)),
            pl.BlockSpec((D, 64), lambda i: (0, 0)),
            pl.BlockSpec((64, 1), lambda i: (0, 0)),
        ],
        out_specs=pl.BlockSpec((_BR, 1), lambda i: (i, 0)),
        out_shape=jax.ShapeDtypeStruct((N, 1), jnp.float32),
    )(accp, xwp, dis, b1, W2, fc_W)


def _tc_final(acc2p, zp, dis, b2, fc_W, fc_b):
    def body(acc2p_ref, zp_ref, dis_ref, b2_ref, fcw_ref, fcb_ref, out_ref):
        t = dis_ref[...] * (acc2p_ref[0] + acc2p_ref[1] + zp_ref[...])
        bias = jnp.dot(b2_ref[...], fcw_ref[...],
                       preferred_element_type=jnp.float32) + fcb_ref[...]
        out_ref[...] = jax.nn.sigmoid(t + bias)

    return pl.pallas_call(
        body,
        grid=(_GRID,),
        in_specs=[
            pl.BlockSpec((NC, _BR, 1), lambda i: (0, i, 0)),
            pl.BlockSpec((_BR, 1), lambda i: (i, 0)),
            pl.BlockSpec((_BR, 1), lambda i: (i, 0)),
            pl.BlockSpec((1, 64), lambda i: (0, 0)),
            pl.BlockSpec((64, 1), lambda i: (0, 0)),
            pl.BlockSpec((1, 1), lambda i: (0, 0)),
        ],
        out_specs=pl.BlockSpec((_BR, 1), lambda i: (i, 0)),
        out_shape=jax.ShapeDtypeStruct((N, 1), jnp.float32),
    )(acc2p, zp, dis, b2, fc_W, fc_b)


def kernel(x, edge_index, W1, b1, W2, b2, fc_W, fc_b):
    srcm = edge_index[0].astype(jnp.int32).reshape(NCH, B)
    dstm = edge_index[1].astype(jnp.int32).reshape(NCH, B)

    zeros2d = jnp.zeros((RPT, D), jnp.float32)
    zeros1d = jnp.zeros((RPT,), jnp.float32)
    ones_b = jnp.ones((B,), jnp.float32)
    x = x.astype(jnp.float32)

    xw = _tc_matmul(x, W1)
    degp = _make_deg_pass()(dstm, ones_b, zeros1d)
    degp = degp.reshape(NC, NRA, 1)

    xwp, dis = _tc_scale(xw, degp)

    accp = _make_edge_pass(D, stage_table=False, nbuf=2)(xwp, srcm, dstm,
                                                         zeros2d)

    zp = _tc_layer2(accp, xwp, dis, b1.reshape(1, D), W2, fc_W)

    zp_pad = jnp.concatenate([zp.reshape(N), jnp.zeros((NRA - N,), jnp.float32)])
    acc2p = _make_edge_pass(1, stage_table=True, nbuf=4)(zp_pad, srcm, dstm,
                                                         zeros1d)
    acc2p = acc2p.reshape(NC, NRA, 1)

    return _tc_final(acc2p, zp, dis, b2.reshape(1, 64), fc_W,
                     fc_b.reshape(1, 1))


# speedup vs baseline: 1.0321x; 1.0067x over previous
import functools

import jax
import jax.numpy as jnp
from jax import lax
from jax.experimental import pallas as pl
from jax.experimental.pallas import tpu as pltpu
from jax.experimental.pallas import tpu_sc as plsc

N = 10000
E = 320000
D = 128
NRA = 10240

NC = 2
NS = 16
NW = NC * NS
B = 125
KB = 8
CPT = 80
NG = CPT // KB
NG2 = NG // 2
NCH = NW * CPT
RPT = NRA // NS

_MESH = plsc.VectorSubcoreMesh(core_axis_name="c", subcore_axis_name="s")


def _worker_id():
    return lax.axis_index("s") * NC + lax.axis_index("c")


def _gather_scatter_2groups(table, sb0, db0, sb1, db1, acc_sh, vals, gsem,
                            ssem, at_idx_wait, at_prefetch):
    nbuf = len(vals)
    n = 2 * KB
    h_g = [None] * nbuf
    h_s = [None] * nbuf
    state = {"started": 0, "waited": False}

    def start_one():
        j = state["started"]
        if j >= KB and not state["waited"]:
            at_idx_wait()
            state["waited"] = True
        sb = sb0 if j < KB else sb1
        p = j % nbuf
        if h_s[p] is not None:
            h_s[p].wait()
        h_g[p] = pltpu.async_copy(table.at[sb.at[j % KB]], vals[p], gsem[p])
        state["started"] = j + 1

    for _ in range(min(nbuf, n)):
        start_one()
    for k in range(n):
        db = db0 if k < KB else db1
        p = k % nbuf
        h_g[p].wait()
        if k == KB - 1:
            at_prefetch()
        h_s[p] = pltpu.async_copy(vals[p], acc_sh.at[db.at[k % KB]], ssem[p],
                                  add=True)
        if state["started"] < n:
            start_one()
    for p in range(nbuf):
        if h_s[p] is not None:
            h_s[p].wait()


def _make_edge_pass(d_feat, stage_table, nbuf):
    if d_feat == 1:
        tshape = (NRA,)
        ashape = (NRA,)
        vshape = (B,)
        zshape = (RPT,)
    else:
        tshape = (N, d_feat)
        ashape = (NRA, d_feat)
        vshape = (B, d_feat)
        zshape = (RPT, d_feat)

    scratch = (
        [pltpu.VMEM((KB, B), jnp.int32)] * 4
        + [pltpu.VMEM(vshape, jnp.float32)] * nbuf
        + [pltpu.VMEM_SHARED(ashape, jnp.float32)]
        + ([pltpu.VMEM_SHARED(tshape, jnp.float32)] if stage_table else [])
        + [pltpu.SemaphoreType.DMA] * (2 * nbuf + 2)
    )

    @functools.partial(
        pl.kernel,
        mesh=_MESH,
        out_type=jax.ShapeDtypeStruct((NC,) + ashape, jnp.float32),
        scratch_types=scratch,
    )
    def edge_pass(table_hbm, srcm_hbm, dstm_hbm, zeros_hbm, out_hbm,
                  sb0, sb1, db0, db1, *rest):
        vals = tuple(rest[:nbuf])
        acc_sh = rest[nbuf]
        rest = rest[nbuf + 1:]
        if stage_table:
            tab_sh = rest[0]
            rest = rest[1:]
        else:
            tab_sh = None
        gsem = tuple(rest[:nbuf])
        ssem = tuple(rest[nbuf:2 * nbuf])
        semi0, semi1 = rest[2 * nbuf:]

        c = lax.axis_index("c")
        s = lax.axis_index("s")
        wid = _worker_id()
        r0 = s * RPT
        row0 = wid * CPT

        pltpu.async_copy(srcm_hbm.at[pl.ds(row0, KB)], sb0, semi0)
        pltpu.async_copy(dstm_hbm.at[pl.ds(row0, KB)], db0, semi0)
        pltpu.sync_copy(zeros_hbm, acc_sh.at[pl.ds(r0, RPT)])
        if stage_table:
            pltpu.sync_copy(table_hbm.at[pl.ds(r0, RPT)],
                            tab_sh.at[pl.ds(r0, RPT)])
            table = tab_sh
        else:
            table = table_hbm
        plsc.subcore_barrier()

        def body(t, carry):
            grow1 = row0 + (2 * t + 1) * KB
            hi_s = pltpu.async_copy(srcm_hbm.at[pl.ds(grow1, KB)], sb1, semi1)
            hi_d = pltpu.async_copy(dstm_hbm.at[pl.ds(grow1, KB)], db1, semi1)
            pltpu.make_async_copy(srcm_hbm.at[pl.ds(row0, KB)], sb0, semi0).wait()
            pltpu.make_async_copy(dstm_hbm.at[pl.ds(row0, KB)], db0, semi0).wait()

            def at_idx_wait():
                hi_s.wait()
                hi_d.wait()

            def at_prefetch():
                @pl.when(t < NG2 - 1)
                def _():
                    grow2 = row0 + (2 * t + 2) * KB
                    pltpu.async_copy(srcm_hbm.at[pl.ds(grow2, KB)], sb0, semi0)
                    pltpu.async_copy(dstm_hbm.at[pl.ds(grow2, KB)], db0, semi0)

            _gather_scatter_2groups(table, sb0, db0, sb1, db1, acc_sh, vals,
                                    gsem, ssem, at_idx_wait, at_prefetch)
            return carry

        lax.fori_loop(0, NG2, body, 0)
        plsc.subcore_barrier()
        pltpu.sync_copy(acc_sh.at[pl.ds(r0, RPT)],
                        out_hbm.at[c].at[pl.ds(r0, RPT)])

    return edge_pass


def _make_deg_pass():
    @functools.partial(
        pl.kernel,
        mesh=_MESH,
        out_type=jax.ShapeDtypeStruct((NC, NRA), jnp.float32),
        scratch_types=[
            pltpu.VMEM((KB, B), jnp.int32),
            pltpu.VMEM((KB, B), jnp.int32),
            pltpu.VMEM((B,), jnp.float32),
            pltpu.VMEM_SHARED((NRA,), jnp.float32),
            pltpu.SemaphoreType.DMA,
            pltpu.SemaphoreType.DMA,
            pltpu.SemaphoreType.DMA,
        ],
    )
    def deg_pass(dstm_hbm, ones_hbm, zeros_hbm, out_hbm, db0, db1, ones_v,
                 acc_sh, ssem, semi0, semi1):
        c = lax.axis_index("c")
        s = lax.axis_index("s")
        wid = _worker_id()
        r0 = s * RPT
        pltpu.sync_copy(zeros_hbm, acc_sh.at[pl.ds(r0, RPT)])
        pltpu.sync_copy(ones_hbm, ones_v)
        plsc.subcore_barrier()

        row0 = wid * CPT

        def scatter_group(db):
            hs = [pltpu.async_copy(ones_v, acc_sh.at[db.at[k]], ssem, add=True)
                  for k in range(KB)]
            for h in hs:
                h.wait()

        pltpu.async_copy(dstm_hbm.at[pl.ds(row0, KB)], db0, semi0)

        def body(t, carry):
            grow1 = row0 + (2 * t + 1) * KB
            hi_d = pltpu.async_copy(dstm_hbm.at[pl.ds(grow1, KB)], db1, semi1)
            pltpu.make_async_copy(dstm_hbm.at[pl.ds(row0, KB)], db0, semi0).wait()
            scatter_group(db0)

            @pl.when(t < NG2 - 1)
            def _():
                grow2 = row0 + (2 * t + 2) * KB
                pltpu.async_copy(dstm_hbm.at[pl.ds(grow2, KB)], db0, semi0)

            hi_d.wait()
            scatter_group(db1)
            return carry

        lax.fori_loop(0, NG2, body, 0)
        plsc.subcore_barrier()
        pltpu.sync_copy(acc_sh.at[pl.ds(r0, RPT)],
                        out_hbm.at[c].at[pl.ds(r0, RPT)])

    return deg_pass


_BR = 1000
_GRID = N // _BR


def _tc_matmul(x, W1):
    def body(x_ref, w_ref, xw_ref):
        xw_ref[...] = jnp.dot(x_ref[...], w_ref[...],
                              preferred_element_type=jnp.float32)

    return pl.pallas_call(
        body,
        grid=(_GRID,),
        in_specs=[
            pl.BlockSpec((_BR, D), lambda i: (i, 0)),
            pl.BlockSpec((D, D), lambda i: (0, 0)),
        ],
        out_specs=pl.BlockSpec((_BR, D), lambda i: (i, 0)),
        out_shape=jax.ShapeDtypeStruct((N, D), jnp.float32),
    )(x, W1)


def _tc_scale(xw, degp):
    def body(xw_ref, degp_ref, xwp_ref, dis_ref):
        deg = degp_ref[0] + degp_ref[1] + 1.0
        dis = lax.rsqrt(deg)
        xwp_ref[...] = dis * xw_ref[...]
        dis_ref[...] = dis

    return pl.pallas_call(
        body,
        grid=(_GRID,),
        in_specs=[
            pl.BlockSpec((_BR, D), lambda i: (i, 0)),
            pl.BlockSpec((NC, _BR, 1), lambda i: (0, i, 0)),
        ],
        out_specs=[
            pl.BlockSpec((_BR, D), lambda i: (i, 0)),
            pl.BlockSpec((_BR, 1), lambda i: (i, 0)),
        ],
        out_shape=[
            jax.ShapeDtypeStruct((N, D), jnp.float32),
            jax.ShapeDtypeStruct((N, 1), jnp.float32),
        ],
    )(xw, degp)


def _tc_layer2(accp, xwp, dis, b1, W2, fc_W):
    def body(accp_ref, xwp_ref, dis_ref, b1_ref, w2_ref, fcw_ref, zp_ref):
        acc = accp_ref[0] + accp_ref[1] + xwp_ref[...]
        h1 = jnp.maximum(dis_ref[...] * acc + b1_ref[...], 0.0)
        w2f = jnp.dot(w2_ref[...], fcw_ref[...], preferred_element_type=jnp.float32)
        z = jnp.dot(h1, w2f, preferred_element_type=jnp.float32)
        zp_ref[...] = dis_ref[...] * z

    return pl.pallas_call(
        body,
        grid=(_GRID,),
        in_specs=[
            pl.BlockSpec((NC, _BR, D), lambda i: (0, i, 0)),
            pl.BlockSpec((_BR, D), lambda i: (i, 0)),
            pl.BlockSpec((_BR, 1), lambda i: (i, 0)),
            pl.BlockSpec((1, D), lambda i: (0, 0)),
            pl.BlockSpec((D, 64), lambda i: (0, 0)),
            pl.BlockSpec((64, 1), lambda i: (0, 0)),
        ],
        out_specs=pl.BlockSpec((_BR, 1), lambda i: (i, 0)),
        out_shape=jax.ShapeDtypeStruct((N, 1), jnp.float32),
    )(accp, xwp, dis, b1, W2, fc_W)


def _tc_final(acc2p, zp, dis, b2, fc_W, fc_b):
    def body(acc2p_ref, zp_ref, dis_ref, b2_ref, fcw_ref, fcb_ref, out_ref):
        t = dis_ref[...] * (acc2p_ref[0] + acc2p_ref[1] + zp_ref[...])
        bias = jnp.dot(b2_ref[...], fcw_ref[...],
                       preferred_element_type=jnp.float32) + fcb_ref[...]
        out_ref[...] = jax.nn.sigmoid(t + bias)

    return pl.pallas_call(
        body,
        grid=(_GRID,),
        in_specs=[
            pl.BlockSpec((NC, _BR, 1), lambda i: (0, i, 0)),
            pl.BlockSpec((_BR, 1), lambda i: (i, 0)),
            pl.BlockSpec((_BR, 1), lambda i: (i, 0)),
            pl.BlockSpec((1, 64), lambda i: (0, 0)),
            pl.BlockSpec((64, 1), lambda i: (0, 0)),
            pl.BlockSpec((1, 1), lambda i: (0, 0)),
        ],
        out_specs=pl.BlockSpec((_BR, 1), lambda i: (i, 0)),
        out_shape=jax.ShapeDtypeStruct((N, 1), jnp.float32),
    )(acc2p, zp, dis, b2, fc_W, fc_b)


def kernel(x, edge_index, W1, b1, W2, b2, fc_W, fc_b):
    srcm = edge_index[0].astype(jnp.int32).reshape(NCH, B)
    dstm = edge_index[1].astype(jnp.int32).reshape(NCH, B)

    zeros2d = jnp.zeros((RPT, D), jnp.float32)
    zeros1d = jnp.zeros((RPT,), jnp.float32)
    ones_b = jnp.ones((B,), jnp.float32)
    x = x.astype(jnp.float32)

    xw = _tc_matmul(x, W1)
    degp = _make_deg_pass()(dstm, ones_b, zeros1d)
    degp = degp.reshape(NC, NRA, 1)

    xwp, dis = _tc_scale(xw, degp)

    accp = _make_edge_pass(D, stage_table=False, nbuf=2)(xwp, srcm, dstm,
                                                         zeros2d)

    zp = _tc_layer2(accp, xwp, dis, b1.reshape(1, D), W2, fc_W)

    zp_pad = jnp.concatenate([zp.reshape(N), jnp.zeros((NRA - N,), jnp.float32)])
    acc2p = _make_edge_pass(1, stage_table=True, nbuf=4)(zp_pad, srcm, dstm,
                                                         zeros1d)
    acc2p = acc2p.reshape(NC, NRA, 1)

    return _tc_final(acc2p, zp, dis, b2.reshape(1, 64), fc_W,
                     fc_b.reshape(1, 1))
